# baseline (speedup 1.0000x reference)
import functools

import jax
import jax.numpy as jnp
from jax.experimental import pallas as pl
from jax.experimental.pallas import tpu as pltpu


def _phrase_kernel(s1_hbm, s2_hbm, wenc_ref, benc_ref, w1_ref, b1_ref,
                   w2_ref, b2_ref, out_ref, buf1, buf2, sems,
                   *, odim, bt, lc, nchunks):
    p = pl.program_id(0)
    b0 = p * bt

    copies = []
    for k in range(nchunks):
        c1 = pltpu.make_async_copy(
            s1_hbm.at[pl.ds(k * lc, lc), pl.ds(b0, bt), :],
            buf1.at[pl.ds(k * lc, lc)],
            sems.at[0, k])
        c1.start()
        c2 = pltpu.make_async_copy(
            s2_hbm.at[pl.ds(k * lc, lc), pl.ds(b0, bt), :],
            buf2.at[pl.ds(k * lc, lc)],
            sems.at[1, k])
        c2.start()
        copies.append((c1, c2))

    acc1 = None
    acc2 = None
    for k in range(nchunks):
        copies[k][0].wait()
        s1 = jnp.sum(buf1[pl.ds(k * lc, lc)], axis=0)
        acc1 = s1 if acc1 is None else acc1 + s1
        copies[k][1].wait()
        s2 = jnp.sum(buf2[pl.ds(k * lc, lc)], axis=0)
        acc2 = s2 if acc2 is None else acc2 + s2

    wenc = wenc_ref[...]
    benc = benc_ref[...]
    h1 = jnp.tanh(jnp.dot(acc1, wenc,
                          preferred_element_type=jnp.float32) + benc)
    h2 = jnp.tanh(jnp.dot(acc2, wenc,
                          preferred_element_type=jnp.float32) + benc)

    w1 = w1_ref[...]
    z = (jnp.dot(h1, w1[0 * odim:1 * odim, :],
                 preferred_element_type=jnp.float32)
         + jnp.dot(h2, w1[1 * odim:2 * odim, :],
                   preferred_element_type=jnp.float32)
         + jnp.dot(jnp.abs(h1 - h2), w1[2 * odim:3 * odim, :],
                   preferred_element_type=jnp.float32)
         + jnp.dot(h1 * h2, w1[3 * odim:4 * odim, :],
                   preferred_element_type=jnp.float32)
         + b1_ref[...])
    z = jnp.maximum(z, 0.0)

    logits = jnp.sum(z * w2_ref[...], axis=-1) + b2_ref[0]
    out_ref[...] = (1.0 / (1.0 + jnp.exp(-logits)))[None, :]


def kernel(seq1, seq2, wenc, benc, w1, b1, w2, b2):
    L, B, idim = seq1.shape
    odim = wenc.shape[1]

    bt = B if B <= 512 else 512
    assert B % bt == 0
    nb = B // bt

    lc = L
    for cand in (8, 4, 2, 1):
        if L % cand == 0:
            lc = cand
            break
    nchunks = L // lc

    wenc_scaled = wenc * (1.0 / L)
    w2_row = w2.reshape(1, odim)
    b2_s = b2.reshape(1)

    const = lambda shape: pl.BlockSpec(shape, lambda b: (0, 0))

    out = pl.pallas_call(
        functools.partial(_phrase_kernel, odim=odim, bt=bt, lc=lc,
                          nchunks=nchunks),
        out_shape=jax.ShapeDtypeStruct((1, B), jnp.float32),
        grid=(nb,),
        in_specs=[
            pl.BlockSpec(memory_space=pl.ANY),
            pl.BlockSpec(memory_space=pl.ANY),
            const((idim, odim)),
            const((1, odim)),
            const((4 * odim, odim)),
            const((1, odim)),
            const((1, odim)),
            pl.BlockSpec(memory_space=pltpu.MemorySpace.SMEM),
        ],
        out_specs=pl.BlockSpec((1, bt), lambda b: (0, b)),
        scratch_shapes=[
            pltpu.VMEM((L, bt, idim), jnp.float32),
            pltpu.VMEM((L, bt, idim), jnp.float32),
            pltpu.SemaphoreType.DMA((2, nchunks)),
        ],
        compiler_params=pltpu.CompilerParams(
            dimension_semantics=("parallel",),
            vmem_limit_bytes=56 << 20),
    )(seq1, seq2, wenc_scaled, benc, w1, b1, w2_row, b2_s)

    return out.reshape(B, 1)
